# baseline (speedup 1.0000x reference)
import functools

import jax
import jax.numpy as jnp
from jax import lax
from jax.experimental import pallas as pl
from jax.experimental.pallas import tpu as pltpu
from jax.experimental.pallas import tpu_sc as plsc

NFEAT = 1000000
NEMB = 16
B = 16384
F = 26
N = B * F
NW = 32
PW = N // NW
C = 1664
NCHUNK = PW // C

_mesh = plsc.VectorSubcoreMesh(core_axis_name="c", subcore_axis_name="s")


@functools.partial(
    pl.kernel,
    out_type=jax.ShapeDtypeStruct((N, NEMB), jnp.float32),
    mesh=_mesh,
    compiler_params=pltpu.CompilerParams(use_tc_tiling_on_sc=False),
    scratch_types=[
        pltpu.VMEM((PW,), jnp.int32),
        pltpu.VMEM((PW,), jnp.float32),
        pltpu.VMEM((C, NEMB), jnp.float32),
        pltpu.SemaphoreType.DMA,
    ],
)
def _emb_lookup(ids_hbm, vals_hbm, table_hbm, out_hbm, ids_v, vals_v, rows_v, sem):
    wid = lax.axis_index("s") * 2 + lax.axis_index("c")
    base = wid * PW
    pltpu.sync_copy(ids_hbm.at[pl.ds(base, PW)], ids_v)
    pltpu.sync_copy(vals_hbm.at[pl.ds(base, PW)], vals_v)

    def chunk_body(k, carry):
        off = pl.multiple_of(k * C, 8)
        pltpu.async_copy(table_hbm.at[ids_v.at[pl.ds(off, C)]], rows_v, sem).wait()

        def grp_body(g, carry2):
            vv = vals_v[pl.ds(off + g * NEMB, NEMB)]
            for j in range(NEMB):
                r = g * NEMB + j
                rows_v[r, :] = rows_v[r, :] * vv[j]
            return carry2

        lax.fori_loop(0, C // NEMB, grp_body, 0)
        pltpu.sync_copy(rows_v, out_hbm.at[pl.ds(base + off, C)])
        return carry

    lax.fori_loop(0, NCHUNK, chunk_body, 0)


def kernel(ids, vals, table):
    ids_flat = ids.reshape(N).astype(jnp.int32)
    vals_flat = vals.reshape(N)
    out = _emb_lookup(ids_flat, vals_flat, table)
    return out.reshape(B, F, NEMB)


# speedup vs baseline: 1.2892x; 1.2892x over previous
import functools

import jax
import jax.numpy as jnp
from jax import lax
from jax.experimental import pallas as pl
from jax.experimental.pallas import tpu as pltpu
from jax.experimental.pallas import tpu_sc as plsc

NFEAT = 1000000
NEMB = 16
B = 16384
F = 26
NW = 32
RW = B // NW
RC = 64
NCHUNK = RW // RC
CF = RC * F

_mesh = plsc.VectorSubcoreMesh(core_axis_name="c", subcore_axis_name="s")


@functools.partial(
    pl.kernel,
    out_type=jax.ShapeDtypeStruct((B, F, NEMB), jnp.float32),
    mesh=_mesh,
    compiler_params=pltpu.CompilerParams(use_tc_tiling_on_sc=False, needs_layout_passes=False),
    scratch_types=[
        pltpu.VMEM((RW, F), jnp.int32),
        pltpu.VMEM((RW, F), jnp.float32),
        pltpu.VMEM((RW * F,), jnp.int32),
        pltpu.VMEM((CF, NEMB), jnp.float32),
        pltpu.VMEM((RC, F, NEMB), jnp.float32),
        pltpu.SemaphoreType.DMA,
    ],
)
def _emb_lookup(ids_hbm, vals_hbm, table_hbm, out_hbm,
                ids_v, vals_v, idx_v, rows_v, outc_v, sem):
    wid = lax.axis_index("s") * 2 + lax.axis_index("c")
    b0 = wid * RW
    pltpu.sync_copy(ids_hbm.at[pl.ds(b0, RW), :], ids_v)
    pltpu.sync_copy(vals_hbm.at[pl.ds(b0, RW), :], vals_v)

    lanes = lax.iota(jnp.int32, 16)

    def flat_body(i, carry):
        a0 = ids_v[i, pl.ds(0, 16)]
        a1 = ids_v[i, pl.ds(F - 16, 16)]
        plsc.store_scatter(idx_v, [i * F + lanes], a0)
        plsc.store_scatter(idx_v, [i * F + (F - 16) + lanes], a1)
        return carry

    lax.fori_loop(0, RW, flat_body, 0)

    def chunk_body(k, carry):
        off = pl.multiple_of(k * CF, 8)
        pltpu.async_copy(table_hbm.at[idx_v.at[pl.ds(off, CF)]], rows_v, sem).wait()

        def row_body(i, carry2):
            vv0 = vals_v[k * RC + i, pl.ds(0, 16)]
            vv1 = vals_v[k * RC + i, pl.ds(F - 16, 16)]
            for j in range(F):
                v = vv0[j] if j < 16 else vv1[j - (F - 16)]
                outc_v[i, j, :] = rows_v[i * F + j, :] * v
            return carry2

        lax.fori_loop(0, RC, row_body, 0)
        pltpu.sync_copy(outc_v, out_hbm.at[pl.ds(b0 + k * RC, RC), :, :])
        return carry

    lax.fori_loop(0, NCHUNK, chunk_body, 0)


def kernel(ids, vals, table):
    return _emb_lookup(ids.astype(jnp.int32), vals, table)


# speedup vs baseline: 1.4203x; 1.1017x over previous
import functools

import jax
import jax.numpy as jnp
from jax import lax
from jax.experimental import pallas as pl
from jax.experimental.pallas import tpu as pltpu
from jax.experimental.pallas import tpu_sc as plsc

NFEAT = 1000000
NEMB = 16
B = 16384
F = 26
NW = 32
RW = B // NW
RC = 64
NCHUNK = RW // RC
CF = RC * F

_mesh = plsc.VectorSubcoreMesh(core_axis_name="c", subcore_axis_name="s")


@functools.partial(
    pl.kernel,
    out_type=jax.ShapeDtypeStruct((F, NEMB, B), jnp.float32),
    mesh=_mesh,
    compiler_params=pltpu.CompilerParams(use_tc_tiling_on_sc=False, needs_layout_passes=False),
    scratch_types=[
        pltpu.VMEM((F, RW), jnp.int32),
        pltpu.VMEM((F, RW), jnp.float32),
        pltpu.VMEM((RW * F,), jnp.int32),
        pltpu.VMEM((CF, NEMB), jnp.float32),
        pltpu.VMEM((F, NEMB, RC), jnp.float32),
        pltpu.SemaphoreType.DMA,
    ],
)
def _emb_lookup(ids_hbm, vals_hbm, table_hbm, out_hbm,
                ids_v, vals_v, idx_v, rows_v, outc_v, sem):
    wid = lax.axis_index("s") * 2 + lax.axis_index("c")
    b0 = wid * RW
    pltpu.sync_copy(ids_hbm.at[:, pl.ds(b0, RW)], ids_v)
    pltpu.sync_copy(vals_hbm.at[:, pl.ds(b0, RW)], vals_v)

    lanes = lax.iota(jnp.int32, 16)
    lanes_hi = lanes + (F - 16)

    def flat_body(i, carry):
        a0 = plsc.load_gather(ids_v, [lanes, jnp.full((16,), i, jnp.int32)])
        a1 = plsc.load_gather(ids_v, [lanes_hi, jnp.full((16,), i, jnp.int32)])
        plsc.store_scatter(idx_v, [i * F + lanes], a0)
        plsc.store_scatter(idx_v, [i * F + (F - 16) + lanes], a1)
        return carry

    lax.fori_loop(0, RW, flat_body, 0)

    def chunk_body(k, carry):
        off = pl.multiple_of(k * CF, 8)
        pltpu.async_copy(table_hbm.at[idx_v.at[pl.ds(off, CF)]], rows_v, sem).wait()

        def row_body(i, carry2):
            bcol = jnp.full((16,), i, jnp.int32)
            vv0 = plsc.load_gather(vals_v, [lanes, k * RC + bcol])
            vv1 = plsc.load_gather(vals_v, [lanes_hi, k * RC + bcol])
            for j in range(F):
                v = vv0[j] if j < 16 else vv1[j - (F - 16)]
                plsc.store_scatter(
                    outc_v,
                    [jnp.full((16,), j, jnp.int32), lanes, bcol],
                    rows_v[i * F + j, :] * v,
                )
            return carry2

        lax.fori_loop(0, RC, row_body, 0)
        pltpu.sync_copy(outc_v, out_hbm.at[:, :, pl.ds(b0 + k * RC, RC)])
        return carry

    lax.fori_loop(0, NCHUNK, chunk_body, 0)


def kernel(ids, vals, table):
    out_t = _emb_lookup(ids.astype(jnp.int32).T, vals.T, table)
    return jnp.transpose(out_t, (2, 0, 1))
